# Initial kernel scaffold; baseline (speedup 1.0000x reference)
import jax
import jax.numpy as jnp
from jax.experimental import pallas as pl


def kernel(images, features, conv_w, conv_b, cls_w, cls_b, reg_w, reg_b):
    raise NotImplementedError("write your pallas kernel here")


# speedup vs baseline: 5.8155x; 5.8155x over previous
import math

import numpy as np
import jax
import jax.numpy as jnp
from jax import lax
from jax.experimental import pallas as pl
from jax.experimental.pallas import tpu as pltpu

_GRID = 100
_PAD = _GRID + 2
_NPIX = _PAD * _PAD
_Q0 = _PAD + 1
_QLEN = _NPIX - 2 * _Q0
_CIN = 256
_NA = 9
_PRE = 2000
_POST = 1000
_THR = 0.7
_CLIP = math.log(1000.0 / 16)


def _anchor_consts():
    area = np.array((128.0, 256.0, 512.0), dtype=np.float32)
    r = np.array((0.5, 1.0, 2.0), dtype=np.float32)
    wr = np.sqrt(r)
    hr = 1.0 / wr
    widths = (wr[:, None] * area[None, :]).reshape(-1)
    heights = (hr[:, None] * area[None, :]).reshape(-1)
    base = np.round(np.stack([-widths, -heights, widths, heights], axis=1) / 2.0)
    w9 = (base[:, 2] - base[:, 0]).astype(np.float32)
    h9 = (base[:, 3] - base[:, 1]).astype(np.float32)
    cxo = (base[:, 0] + 0.5 * w9).astype(np.float32)
    cyo = (base[:, 1] + 0.5 * h9).astype(np.float32)
    return w9, h9, cxo, cyo


_W9, _H9, _CXO, _CYO = _anchor_consts()


def _head_kernel(x_ref, wt_ref, cb_ref, wh_ref, bh_ref, anc_ref,
                 out_ref, acc_ref):
    for t in range(9):
        dh, dw = t // 3, t % 3
        off = (dh - 1) * _PAD + (dw - 1)
        y = jnp.dot(x_ref[0, pl.ds(_Q0 + off, _QLEN), :], wt_ref[t],
                    preferred_element_type=jnp.float32)
        if t == 0:
            acc_ref[pl.ds(_Q0, _QLEN), :] = y
        else:
            acc_ref[pl.ds(_Q0, _QLEN), :] += y
    acc_ref[pl.ds(0, _Q0), :] = jnp.zeros((_Q0, _CIN), jnp.float32)
    acc_ref[pl.ds(_NPIX - _Q0, _Q0), :] = jnp.zeros((_Q0, _CIN), jnp.float32)
    acc_ref[pl.ds(_Q0, _QLEN), :] = jnp.maximum(
        acc_ref[pl.ds(_Q0, _QLEN), :] + cb_ref[0, :], 0.0)
    yh = jnp.dot(acc_ref[pl.ds(0, _NPIX), :], wh_ref[...],
                 preferred_element_type=jnp.float32) + bh_ref[0, :]
    obj = yh[:, 0:_NA]
    dxv = yh[:, _NA:2 * _NA]
    dyv = yh[:, 2 * _NA:3 * _NA]
    dwv = yh[:, 3 * _NA:4 * _NA]
    dhv = yh[:, 4 * _NA:5 * _NA]
    q = lax.broadcasted_iota(jnp.int32, (_NPIX, _NA), 0)
    hrow = (q // _PAD - 1).astype(jnp.float32)
    wcol = (q % _PAD - 1).astype(jnp.float32)
    wv = anc_ref[0, pl.ds(0, 1), :]
    hv = anc_ref[0, pl.ds(1, 1), :]
    cx = 8.0 * hrow + anc_ref[0, pl.ds(2, 1), :]
    cy = 8.0 * wcol + anc_ref[0, pl.ds(3, 1), :]
    dwc = jnp.minimum(dwv, _CLIP)
    dhc = jnp.minimum(dhv, _CLIP)
    pcx = dxv * wv + cx
    pcy = dyv * hv + cy
    pw = jnp.exp(dwc) * wv
    ph = jnp.exp(dhc) * hv
    out_ref[0] = jnp.concatenate([
        obj,
        jnp.clip(pcx - 0.5 * pw, 0.0, 800.0),
        jnp.clip(pcy - 0.5 * ph, 0.0, 800.0),
        jnp.clip(pcx + 0.5 * pw, 0.0, 800.0),
        jnp.clip(pcy + 0.5 * ph, 0.0, 800.0)], axis=1)


_R = 8
_C = _PRE // _R


def _nms_kernel(x1_ref, y1_ref, x2_ref, y2_ref, out_ref, keep_ref):
    x1v = x1_ref[0]
    y1v = y1_ref[0]
    x2v = x2_ref[0]
    y2v = y2_ref[0]
    area = (x2v - x1v) * (y2v - y1v)
    subl = lax.broadcasted_iota(jnp.int32, (_R, _C), 0)
    lane = lax.broadcasted_iota(jnp.int32, (_R, _C), 1)
    lin = subl * _C + lane
    validv = jnp.where(
        jnp.logical_and(x2v - x1v >= 1e-3, y2v - y1v >= 1e-3), 1.0, 0.0)
    keep_ref[...] = jnp.ones((_R, _C), jnp.float32)
    out_ref[...] = jnp.zeros((1, _POST, 4), jnp.float32)

    def body(carry):
        i, cnt = carry
        r = i // _C
        c = i - r * _C
        m2d = jnp.logical_and(subl == r, lane == c)

        def ext(v):
            return jnp.sum(jnp.where(m2d, v, 0.0))

        x1s = ext(x1v)
        y1s = ext(y1v)
        x2s = ext(x2v)
        y2s = ext(y2v)
        ki = ext(keep_ref[...])
        vi = ext(validv)

        @pl.when(ki > 0.0)
        def _():
            iw = jnp.maximum(jnp.minimum(x2s, x2v) - jnp.maximum(x1s, x1v), 0.0)
            ih = jnp.maximum(jnp.minimum(y2s, y2v) - jnp.maximum(y1s, y1v), 0.0)
            inter = iw * ih
            ai = (x2s - x1s) * (y2s - y1s)
            iou = inter / (ai + area - inter + 1e-9)
            sup = jnp.logical_and(iou > _THR, lin > i)
            keep_ref[...] = jnp.where(sup, 0.0, keep_ref[...])

        emit = jnp.logical_and(ki > 0.0, vi > 0.0)

        @pl.when(emit)
        def _():
            base = (cnt // 8) * 8
            old = out_ref[0, pl.ds(base, 8), :]
            rowmask = lax.broadcasted_iota(jnp.int32, (8, 4), 0) == (cnt - base)
            lane4 = lax.broadcasted_iota(jnp.int32, (8, 4), 1)
            rowval = jnp.where(lane4 == 0, x1s,
                               jnp.where(lane4 == 1, y1s,
                                         jnp.where(lane4 == 2, x2s, y2s)))
            out_ref[0, pl.ds(base, 8), :] = jnp.where(rowmask, rowval, old)

        return i + 1, cnt + emit.astype(jnp.int32)

    def cond(carry):
        i, cnt = carry
        return jnp.logical_and(i < _PRE, cnt < _POST)

    lax.while_loop(cond, body, (jnp.int32(0), jnp.int32(0)))


def kernel(images, features, conv_w, conv_b, cls_w, cls_b, reg_w, reg_b):
    B = features.shape[0]
    xp = jnp.pad(jnp.transpose(features, (0, 2, 3, 1)),
                 ((0, 0), (1, 1), (1, 1), (0, 0))).reshape(B, _NPIX, _CIN)
    wt = jnp.transpose(conv_w, (2, 3, 1, 0)).reshape(9, _CIN, _CIN)
    cw = cls_w[:, :, 0, 0].T
    rw = reg_w[:, :, 0, 0].reshape(_NA, 4, _CIN)
    wall = jnp.concatenate(
        [cw, rw[:, 0, :].T, rw[:, 1, :].T, rw[:, 2, :].T, rw[:, 3, :].T], axis=1)
    rb = reg_b.reshape(_NA, 4)
    ball = jnp.concatenate([cls_b, rb[:, 0], rb[:, 1], rb[:, 2], rb[:, 3]]
                           ).reshape(1, 5 * _NA)
    cb = conv_b.reshape(1, _CIN)

    dec = pl.pallas_call(
        _head_kernel,
        grid=(B,),
        in_specs=[
            pl.BlockSpec((1, _NPIX, _CIN), lambda b: (b, 0, 0)),
            pl.BlockSpec((9, _CIN, _CIN), lambda b: (0, 0, 0)),
            pl.BlockSpec((1, _CIN), lambda b: (0, 0)),
            pl.BlockSpec((_CIN, 5 * _NA), lambda b: (0, 0)),
            pl.BlockSpec((1, 5 * _NA), lambda b: (0, 0)),
            pl.BlockSpec((1, 4, _NA), lambda b: (0, 0, 0)),
        ],
        out_specs=pl.BlockSpec((1, _NPIX, 5 * _NA), lambda b: (b, 0, 0)),
        out_shape=jax.ShapeDtypeStruct((B, _NPIX, 5 * _NA), jnp.float32),
        scratch_shapes=[pltpu.VMEM((_NPIX, _CIN), jnp.float32)],
    )(xp, wt, cb, wall, ball,
      jnp.asarray(np.stack([_W9, _H9, _CXO, _CYO])).reshape(1, 4, _NA))

    def interior(a):
        return a.reshape(B, _PAD, _PAD, _NA)[:, 1:_GRID + 1, 1:_GRID + 1, :
                                             ].reshape(B, _GRID * _GRID * _NA)

    scores = interior(dec[:, :, 0:_NA])
    _, idx = lax.top_k(scores, _PRE)

    def g(a):
        return jnp.take_along_axis(interior(a), idx, axis=1)

    gx1 = g(dec[:, :, _NA:2 * _NA])
    gy1 = g(dec[:, :, 2 * _NA:3 * _NA])
    gx2 = g(dec[:, :, 3 * _NA:4 * _NA])
    gy2 = g(dec[:, :, 4 * _NA:5 * _NA])

    sel = pl.pallas_call(
        _nms_kernel,
        grid=(B,),
        in_specs=[pl.BlockSpec((1, _R, _C), lambda b: (b, 0, 0))] * 4,
        out_specs=pl.BlockSpec((1, _POST, 4), lambda b: (b, 0, 0)),
        out_shape=jax.ShapeDtypeStruct((B, _POST, 4), jnp.float32),
        scratch_shapes=[pltpu.VMEM((_R, _C), jnp.float32)],
    )(gx1.reshape(B, _R, _C), gy1.reshape(B, _R, _C),
      gx2.reshape(B, _R, _C), gy2.reshape(B, _R, _C))
    return sel


# speedup vs baseline: 5.8578x; 1.0073x over previous
import math

import numpy as np
import jax
import jax.numpy as jnp
from jax import lax
from jax.experimental import pallas as pl
from jax.experimental.pallas import tpu as pltpu

_GRID = 100
_PAD = _GRID + 2
_NPIX = _PAD * _PAD
_Q0 = _PAD + 1
_QLEN = _NPIX - 2 * _Q0
_CIN = 256
_NA = 9
_PRE = 2000
_POST = 1000
_THR = 0.7
_CLIP = math.log(1000.0 / 16)


def _anchor_consts():
    area = np.array((128.0, 256.0, 512.0), dtype=np.float32)
    r = np.array((0.5, 1.0, 2.0), dtype=np.float32)
    wr = np.sqrt(r)
    hr = 1.0 / wr
    widths = (wr[:, None] * area[None, :]).reshape(-1)
    heights = (hr[:, None] * area[None, :]).reshape(-1)
    base = np.round(np.stack([-widths, -heights, widths, heights], axis=1) / 2.0)
    w9 = (base[:, 2] - base[:, 0]).astype(np.float32)
    h9 = (base[:, 3] - base[:, 1]).astype(np.float32)
    cxo = (base[:, 0] + 0.5 * w9).astype(np.float32)
    cyo = (base[:, 1] + 0.5 * h9).astype(np.float32)
    return w9, h9, cxo, cyo


_W9, _H9, _CXO, _CYO = _anchor_consts()


def _head_kernel(x_ref, wt_ref, cb_ref, wh_ref, bh_ref, anc_ref,
                 out_ref, acc_ref):
    for t in range(9):
        dh, dw = t // 3, t % 3
        off = (dh - 1) * _PAD + (dw - 1)
        y = jnp.dot(x_ref[0, pl.ds(_Q0 + off, _QLEN), :], wt_ref[t],
                    preferred_element_type=jnp.float32)
        if t == 0:
            acc_ref[pl.ds(_Q0, _QLEN), :] = y
        else:
            acc_ref[pl.ds(_Q0, _QLEN), :] += y
    acc_ref[pl.ds(0, _Q0), :] = jnp.zeros((_Q0, _CIN), jnp.float32)
    acc_ref[pl.ds(_NPIX - _Q0, _Q0), :] = jnp.zeros((_Q0, _CIN), jnp.float32)
    acc_ref[pl.ds(_Q0, _QLEN), :] = jnp.maximum(
        acc_ref[pl.ds(_Q0, _QLEN), :] + cb_ref[0, :], 0.0)
    yh = jnp.dot(acc_ref[pl.ds(0, _NPIX), :], wh_ref[...],
                 preferred_element_type=jnp.float32) + bh_ref[0, :]
    obj = yh[:, 0:_NA]
    dxv = yh[:, _NA:2 * _NA]
    dyv = yh[:, 2 * _NA:3 * _NA]
    dwv = yh[:, 3 * _NA:4 * _NA]
    dhv = yh[:, 4 * _NA:5 * _NA]
    q = lax.broadcasted_iota(jnp.int32, (_NPIX, _NA), 0)
    hrow = (q // _PAD - 1).astype(jnp.float32)
    wcol = (q % _PAD - 1).astype(jnp.float32)
    wv = anc_ref[0, pl.ds(0, 1), :]
    hv = anc_ref[0, pl.ds(1, 1), :]
    cx = 8.0 * hrow + anc_ref[0, pl.ds(2, 1), :]
    cy = 8.0 * wcol + anc_ref[0, pl.ds(3, 1), :]
    dwc = jnp.minimum(dwv, _CLIP)
    dhc = jnp.minimum(dhv, _CLIP)
    pcx = dxv * wv + cx
    pcy = dyv * hv + cy
    pw = jnp.exp(dwc) * wv
    ph = jnp.exp(dhc) * hv
    out_ref[0] = jnp.concatenate([
        obj,
        jnp.clip(pcx - 0.5 * pw, 0.0, 800.0),
        jnp.clip(pcy - 0.5 * ph, 0.0, 800.0),
        jnp.clip(pcx + 0.5 * pw, 0.0, 800.0),
        jnp.clip(pcy + 0.5 * ph, 0.0, 800.0)], axis=1)


_R = 8
_C = _PRE // _R


def _nms_kernel(x1_ref, y1_ref, x2_ref, y2_ref, x1s_ref, y1s_ref,
                x2s_ref, y2s_ref, vs_ref, out_ref, keep_ref):
    x1v = x1_ref[0]
    y1v = y1_ref[0]
    x2v = x2_ref[0]
    y2v = y2_ref[0]
    area = (x2v - x1v) * (y2v - y1v)
    subl = lax.broadcasted_iota(jnp.int32, (_R, _C), 0)
    lane = lax.broadcasted_iota(jnp.int32, (_R, _C), 1)
    lin = subl * _C + lane
    keep_ref[...] = jnp.ones((_R, _C), jnp.float32)
    out_ref[...] = jnp.zeros((1, _POST, 4), jnp.float32)

    def body(carry):
        i, cnt = carry
        r = i // _C
        c = i - r * _C
        m2d = jnp.logical_and(subl == r, lane == c)
        ki = jnp.sum(jnp.where(m2d, keep_ref[...], 0.0))
        vi = vs_ref[0, 0, i]
        x1s = x1s_ref[0, 0, i]
        y1s = y1s_ref[0, 0, i]
        x2s = x2s_ref[0, 0, i]
        y2s = y2s_ref[0, 0, i]

        @pl.when(ki > 0.0)
        def _():
            iw = jnp.maximum(jnp.minimum(x2s, x2v) - jnp.maximum(x1s, x1v), 0.0)
            ih = jnp.maximum(jnp.minimum(y2s, y2v) - jnp.maximum(y1s, y1v), 0.0)
            inter = iw * ih
            ai = (x2s - x1s) * (y2s - y1s)
            iou = inter / (ai + area - inter + 1e-9)
            sup = jnp.logical_and(iou > _THR, lin > i)
            keep_ref[...] = jnp.where(sup, 0.0, keep_ref[...])

        emit = jnp.logical_and(ki > 0.0, vi > 0.0)

        @pl.when(emit)
        def _():
            base = (cnt // 8) * 8
            old = out_ref[0, pl.ds(base, 8), :]
            rowmask = lax.broadcasted_iota(jnp.int32, (8, 4), 0) == (cnt - base)
            lane4 = lax.broadcasted_iota(jnp.int32, (8, 4), 1)
            rowval = jnp.where(lane4 == 0, x1s,
                               jnp.where(lane4 == 1, y1s,
                                         jnp.where(lane4 == 2, x2s, y2s)))
            out_ref[0, pl.ds(base, 8), :] = jnp.where(rowmask, rowval, old)

        return i + 1, cnt + emit.astype(jnp.int32)

    def cond(carry):
        i, cnt = carry
        return jnp.logical_and(i < _PRE, cnt < _POST)

    lax.while_loop(cond, body, (jnp.int32(0), jnp.int32(0)))


def kernel(images, features, conv_w, conv_b, cls_w, cls_b, reg_w, reg_b):
    B = features.shape[0]
    xp = jnp.pad(jnp.transpose(features, (0, 2, 3, 1)),
                 ((0, 0), (1, 1), (1, 1), (0, 0))).reshape(B, _NPIX, _CIN)
    wt = jnp.transpose(conv_w, (2, 3, 1, 0)).reshape(9, _CIN, _CIN)
    cw = cls_w[:, :, 0, 0].T
    rw = reg_w[:, :, 0, 0].reshape(_NA, 4, _CIN)
    wall = jnp.concatenate(
        [cw, rw[:, 0, :].T, rw[:, 1, :].T, rw[:, 2, :].T, rw[:, 3, :].T], axis=1)
    rb = reg_b.reshape(_NA, 4)
    ball = jnp.concatenate([cls_b, rb[:, 0], rb[:, 1], rb[:, 2], rb[:, 3]]
                           ).reshape(1, 5 * _NA)
    cb = conv_b.reshape(1, _CIN)

    dec = pl.pallas_call(
        _head_kernel,
        grid=(B,),
        in_specs=[
            pl.BlockSpec((1, _NPIX, _CIN), lambda b: (b, 0, 0)),
            pl.BlockSpec((9, _CIN, _CIN), lambda b: (0, 0, 0)),
            pl.BlockSpec((1, _CIN), lambda b: (0, 0)),
            pl.BlockSpec((_CIN, 5 * _NA), lambda b: (0, 0)),
            pl.BlockSpec((1, 5 * _NA), lambda b: (0, 0)),
            pl.BlockSpec((1, 4, _NA), lambda b: (0, 0, 0)),
        ],
        out_specs=pl.BlockSpec((1, _NPIX, 5 * _NA), lambda b: (b, 0, 0)),
        out_shape=jax.ShapeDtypeStruct((B, _NPIX, 5 * _NA), jnp.float32),
        scratch_shapes=[pltpu.VMEM((_NPIX, _CIN), jnp.float32)],
    )(xp, wt, cb, wall, ball,
      jnp.asarray(np.stack([_W9, _H9, _CXO, _CYO])).reshape(1, 4, _NA))

    def interior(a):
        return a.reshape(B, _PAD, _PAD, _NA)[:, 1:_GRID + 1, 1:_GRID + 1, :
                                             ].reshape(B, _GRID * _GRID * _NA)

    scores = interior(dec[:, :, 0:_NA])
    _, idx = lax.top_k(scores, _PRE)

    def g(a):
        return jnp.take_along_axis(interior(a), idx, axis=1)

    gx1 = g(dec[:, :, _NA:2 * _NA])
    gy1 = g(dec[:, :, 2 * _NA:3 * _NA])
    gx2 = g(dec[:, :, 3 * _NA:4 * _NA])
    gy2 = g(dec[:, :, 4 * _NA:5 * _NA])

    validf = jnp.where(
        jnp.logical_and(gx2 - gx1 >= 1e-3, gy2 - gy1 >= 1e-3), 1.0, 0.0)
    smem_spec = pl.BlockSpec((1, 1, _PRE), lambda b: (b, 0, 0),
                             memory_space=pltpu.SMEM)
    sel = pl.pallas_call(
        _nms_kernel,
        grid=(B,),
        in_specs=[pl.BlockSpec((1, _R, _C), lambda b: (b, 0, 0))] * 4
        + [smem_spec] * 5,
        out_specs=pl.BlockSpec((1, _POST, 4), lambda b: (b, 0, 0)),
        out_shape=jax.ShapeDtypeStruct((B, _POST, 4), jnp.float32),
        scratch_shapes=[pltpu.VMEM((_R, _C), jnp.float32)],
    )(gx1.reshape(B, _R, _C), gy1.reshape(B, _R, _C),
      gx2.reshape(B, _R, _C), gy2.reshape(B, _R, _C),
      gx1.reshape(B, 1, _PRE), gy1.reshape(B, 1, _PRE),
      gx2.reshape(B, 1, _PRE), gy2.reshape(B, 1, _PRE),
      validf.reshape(B, 1, _PRE))
    return sel
